# Initial kernel scaffold; baseline (speedup 1.0000x reference)
import jax
import jax.numpy as jnp
from jax.experimental import pallas as pl


def kernel(x, adj1, W00a, b00a, W00b, b00b, W01a, b01a, W01b, b01b, W10a, b10a, W10b, b10b, W11a, b11a, W11b, b11b):
    raise NotImplementedError("write your pallas kernel here")


# speedup vs baseline: 5.5683x; 5.5683x over previous
import jax
import jax.numpy as jnp
from jax import lax
from jax.experimental import pallas as pl
from jax.experimental.pallas import tpu as pltpu

N = 64
D = 64
H = 64
E = 512
EPS = 0.0
F32 = jnp.float32


def _mlp(x, W1, b1, W2, b2):
    h = lax.dot_general(x, W1, (((1,), (0,)), ((), ())),
                        preferred_element_type=F32) + b1
    h = jnp.maximum(h, 0.0)
    return lax.dot_general(h, W2, (((1,), (0,)), ((), ())),
                           preferred_element_type=F32) + b2


def _idgnn_kernel(adjT_ref, x_ref,
                  W00a_ref, b00a_ref, W00b_ref, b00b_ref,
                  W01a_ref, b01a_ref, W01b_ref, b01b_ref,
                  W10a_ref, b10a_ref, W10b_ref, b10b_ref,
                  W11a_ref, b11a_ref, W11b_ref, b11b_ref,
                  out_ref, hp_ref, k2t_ref):
    src = adjT_ref[:, 0:1]
    dst = adjT_ref[:, 1:2]
    iota_e = lax.broadcasted_iota(jnp.int32, (E, N), 1)
    o_src = (src == iota_e).astype(F32)
    o_dst = (dst == iota_e).astype(F32)
    acnt = lax.dot_general(o_src, o_dst, (((0,), (0,)), ((), ())),
                           preferred_element_type=F32)
    A = jnp.minimum(acnt, 1.0)

    k2t_ref[...] = lax.dot_general(A, A, (((0,), (1,)), ((), ())),
                                   preferred_element_type=F32)

    ident = (lax.broadcasted_iota(jnp.int32, (N, N), 0)
             == lax.broadcasted_iota(jnp.int32, (N, N), 1)).astype(F32)
    iota_n = lax.broadcasted_iota(jnp.int32, (N, N), 1).astype(F32)

    x = x_ref[...]

    def scan(Dg, H1):
        def step(t, carry):
            Dg, H1 = carry
            jrow = k2t_ref[pl.ds(t, 1), :]
            jcol = jnp.sum(ident * jrow, axis=1, keepdims=True)
            jc = jnp.minimum(jcol, float(N - 1))
            hit64 = jcol >= float(N) - 0.5
            Mg = (iota_n == jc).astype(F32)
            Ms = Mg * (1.0 - hit64.astype(F32))
            R = lax.dot_general(Mg, A, (((1,), (0,)), ((), ())),
                                preferred_element_type=F32)
            r = jnp.sum(R * ident, axis=1, keepdims=True)
            hp = hp_ref[...]
            P = jnp.sum(R[:, :, None] * hp, axis=1)
            S = P + r * (H1 - Dg)
            hp_ref[...] = jnp.where(Ms[:, :, None] > 0.5, S[:, None, :], hp)
            dghit = jnp.sum(Ms * ident, axis=1, keepdims=True) > 0.5
            Dg = jnp.where(dghit, S, Dg)
            H1 = jnp.where(hit64, S, H1)
            return (Dg, H1)
        return lax.fori_loop(0, N, step, (Dg, H1))

    H0 = _mlp(x, W00a_ref[...], b00a_ref[...], W00b_ref[...], b00b_ref[...])
    H1 = _mlp(x, W01a_ref[...], b01a_ref[...], W01b_ref[...], b01b_ref[...])
    hp_ref[...] = jnp.broadcast_to(H0[None], (N, N, H))
    Dg, _ = scan(H0, H1)
    hj1 = hp_ref[...] + H0[None]
    y_diag = Dg + H0

    h0b = _mlp(hj1.reshape(N * N, D), W10a_ref[...], b10a_ref[...],
               W10b_ref[...], b10b_ref[...]).reshape(N, N, H)
    H1b = _mlp(y_diag, W11a_ref[...], b11a_ref[...],
               W11b_ref[...], b11b_ref[...])
    hp_ref[...] = h0b
    Dg2_init = jnp.sum(ident[:, :, None] * h0b, axis=1)
    Dg2, _ = scan(Dg2_init, H1b)
    out_ref[...] = Dg2 + (1.0 + EPS) * Dg2_init


def kernel(x, adj1, W00a, b00a, W00b, b00b, W01a, b01a, W01b, b01b,
           W10a, b10a, W10b, b10b, W11a, b11a, W11b, b11b):
    adjT = adj1.astype(jnp.int32).T
    biases = [b.reshape(1, H) for b in
              (b00a, b00b, b01a, b01b, b10a, b10b, b11a, b11b)]
    (b00a2, b00b2, b01a2, b01b2, b10a2, b10b2, b11a2, b11b2) = biases
    return pl.pallas_call(
        _idgnn_kernel,
        out_shape=jax.ShapeDtypeStruct((N, H), F32),
        scratch_shapes=[pltpu.VMEM((N, N, H), F32),
                        pltpu.VMEM((N, N), F32)],
    )(adjT, x, W00a, b00a2, W00b, b00b2, W01a, b01a2, W01b, b01b2,
      W10a, b10a2, W10b, b10b2, W11a, b11a2, W11b, b11b2)
